# Initial kernel scaffold; baseline (speedup 1.0000x reference)
import jax
import jax.numpy as jnp
from jax.experimental import pallas as pl


def kernel(x, edge_index, gamma_f, beta_f, W_self_f, W_neigh_f, b_f, gamma_r, beta_r, W_self_r, W_neigh_r, b_r):
    raise NotImplementedError("write your pallas kernel here")


# speedup vs baseline: 3.4389x; 3.4389x over previous
import functools

import jax
import jax.numpy as jnp
from jax import lax
from jax.experimental import pallas as pl
from jax.experimental.pallas import tpu as pltpu
from jax.experimental.pallas import tpu_sc as plsc

N = 10000
E = 320000
D = 128

_NS = 16
_EPW = E // _NS
_K = 64
_Q = 20032
_NCH = _Q // _K
_NP = 10112
_RPT = _NP // _NS
_RFULL = _RPT // _K
_RREM = _RPT - _RFULL * _K

_HIGH = lax.Precision.HIGHEST
_ROWS = 1000


def _prep_body(x_ref, g_ref, b_ref, ws_ref, bb_ref, h_ref, a_ref):
    x = x_ref[...]
    mu = jnp.mean(x, axis=1, keepdims=True)
    xc = x - mu
    var = jnp.mean(xc * xc, axis=1, keepdims=True)
    h = xc * lax.rsqrt(var + 1e-5) * g_ref[0] + b_ref[0]
    h_ref[0] = h
    a_ref[0] = jnp.dot(h, ws_ref[0], precision=_HIGH,
                       preferred_element_type=jnp.float32) + bb_ref[0]


_prep_call = pl.pallas_call(
    _prep_body,
    grid=(2, N // _ROWS),
    in_specs=[
        pl.BlockSpec((_ROWS, D), lambda d, i: (i, 0)),
        pl.BlockSpec((1, 1, D), lambda d, i: (d, 0, 0)),
        pl.BlockSpec((1, 1, D), lambda d, i: (d, 0, 0)),
        pl.BlockSpec((1, D, D), lambda d, i: (d, 0, 0)),
        pl.BlockSpec((1, 1, D), lambda d, i: (d, 0, 0)),
    ],
    out_specs=[
        pl.BlockSpec((1, _ROWS, D), lambda d, i: (d, i, 0)),
        pl.BlockSpec((1, _ROWS, D), lambda d, i: (d, i, 0)),
    ],
    out_shape=[
        jax.ShapeDtypeStruct((2, N, D), jnp.float32),
        jax.ShapeDtypeStruct((2, N, D), jnp.float32),
    ],
)


_sc_mesh = plsc.VectorSubcoreMesh(core_axis_name="c", subcore_axis_name="s")


@functools.partial(
    pl.kernel,
    mesh=_sc_mesh,
    out_type=[
        jax.ShapeDtypeStruct((2, _NP, D), jnp.float32),
        jax.ShapeDtypeStruct((2, _NP, D), jnp.float32),
    ],
    scratch_types=[
        pltpu.VMEM((_K,), jnp.int32),
        pltpu.VMEM((_K,), jnp.int32),
        pltpu.VMEM((_K, D), jnp.float32),
        pltpu.VMEM_SHARED((_NP, D), jnp.float32),
        pltpu.SemaphoreType.DMA,
    ],
)
def _agg(h2_hbm, g_hbm, s_hbm,
         s_out, d_out,
         gidx, sidx, rows, S_sp, sem):
    cid = lax.axis_index("c")
    sid = lax.axis_index("s")
    r0 = sid * _RPT
    rt = r0 + _RFULL * _K
    ebase = (cid * _NS + sid) * _Q

    def _fill_rows(val):
        v16 = jnp.full((16,), val, jnp.float32)

        def _fill_row(r, carry):
            for c in range(D // 16):
                rows[r, pl.ds(c * 16, 16)] = v16
            return carry

        lax.fori_loop(0, _K, _fill_row, 0)

    def _zero_slice():
        for t in range(_RFULL):
            pltpu.sync_copy(rows, S_sp.at[pl.ds(r0 + t * _K, _K)])
        pltpu.sync_copy(rows.at[pl.ds(0, _RREM)], S_sp.at[pl.ds(rt, _RREM)])

    def _export(out):
        for t in range(_RFULL):
            rr = r0 + t * _K
            pltpu.sync_copy(S_sp.at[pl.ds(rr, _K)], rows)
            pltpu.sync_copy(rows, out.at[cid, pl.ds(rr, _K)])
        pltpu.sync_copy(S_sp.at[pl.ds(rt, _RREM)], rows.at[pl.ds(0, _RREM)])
        pltpu.sync_copy(rows.at[pl.ds(0, _RREM)], out.at[cid, pl.ds(rt, _RREM)])

    _fill_rows(0.0)
    _zero_slice()
    plsc.subcore_barrier()

    def _chunk(c, carry):
        b = ebase + c * _K
        pltpu.sync_copy(g_hbm.at[pl.ds(b, _K)], gidx)
        pltpu.sync_copy(s_hbm.at[pl.ds(b, _K)], sidx)
        pltpu.async_copy(h2_hbm.at[gidx], rows, sem).wait()
        pltpu.sync_copy(rows, S_sp.at[sidx], add=True)
        return carry

    lax.fori_loop(0, _NCH, _chunk, 0)
    plsc.subcore_barrier()
    _export(s_out)

    _fill_rows(0.0)
    _zero_slice()
    _fill_rows(1.0)
    plsc.subcore_barrier()

    def _dchunk(c, carry):
        b = ebase + c * _K
        pltpu.sync_copy(s_hbm.at[pl.ds(b, _K)], sidx)
        pltpu.sync_copy(rows, S_sp.at[sidx], add=True)
        return carry

    lax.fori_loop(0, _NCH, _dchunk, 0)
    plsc.subcore_barrier()
    _export(d_out)


def _post_body(x_ref, af_ref, ar_ref, sf_ref, df_ref, sr_ref, dr_ref,
               wnf, wnr, o_ref):
    df = jnp.maximum(df_ref[0][:, 0:1], 1.0)
    dr = jnp.maximum(dr_ref[0][:, 0:1], 1.0)
    nf = sf_ref[0] / df
    nr = sr_ref[0] / dr
    yf = jnp.maximum(
        af_ref[0] + jnp.dot(nf, wnf[...], precision=_HIGH,
                            preferred_element_type=jnp.float32), 0.0)
    yr = jnp.maximum(
        ar_ref[0] + jnp.dot(nr, wnr[...], precision=_HIGH,
                            preferred_element_type=jnp.float32), 0.0)
    o_ref[...] = x_ref[...] + yf + yr


_post_call = pl.pallas_call(
    _post_body,
    grid=(N // _ROWS,),
    in_specs=[
        pl.BlockSpec((_ROWS, D), lambda i: (i, 0)),
        pl.BlockSpec((1, _ROWS, D), lambda i: (0, i, 0)),
        pl.BlockSpec((1, _ROWS, D), lambda i: (1, i, 0)),
        pl.BlockSpec((1, _ROWS, D), lambda i: (0, i, 0)),
        pl.BlockSpec((1, _ROWS, D), lambda i: (0, i, 0)),
        pl.BlockSpec((1, _ROWS, D), lambda i: (1, i, 0)),
        pl.BlockSpec((1, _ROWS, D), lambda i: (1, i, 0)),
        pl.BlockSpec((D, D), lambda i: (0, 0)),
        pl.BlockSpec((D, D), lambda i: (0, 0)),
    ],
    out_specs=pl.BlockSpec((_ROWS, D), lambda i: (i, 0)),
    out_shape=jax.ShapeDtypeStruct((N, D), jnp.float32),
)


def kernel(x, edge_index, gamma_f, beta_f, W_self_f, W_neigh_f, b_f,
           gamma_r, beta_r, W_self_r, W_neigh_r, b_r):
    src = edge_index[0]
    dst = edge_index[1]
    gamma2 = jnp.stack([gamma_f, gamma_r]).reshape(2, 1, D)
    beta2 = jnp.stack([beta_f, beta_r]).reshape(2, 1, D)
    ws2 = jnp.stack([W_self_f.T, W_self_r.T])
    b2 = jnp.stack([b_f, b_r]).reshape(2, 1, D)
    h2, a2 = _prep_call(x, gamma2, beta2, ws2, b2)
    h2_flat = h2.reshape(2 * N, D)
    g3 = jnp.zeros((2, _NS, _Q), jnp.int32)
    g3 = g3.at[:, :, :_EPW].set(
        jnp.stack([src, dst + N]).reshape(2, _NS, _EPW))
    s3 = jnp.full((2, _NS, _Q), N, jnp.int32)
    s3 = s3.at[:, :, :_EPW].set(
        jnp.stack([dst, src]).reshape(2, _NS, _EPW))
    s2, deg2 = _agg(h2_flat, g3.reshape(-1), s3.reshape(-1))
    return _post_call(x, a2, a2, s2, deg2, s2, deg2,
                      W_neigh_f.T, W_neigh_r.T)
